# Initial kernel scaffold; baseline (speedup 1.0000x reference)
import jax
import jax.numpy as jnp
from jax.experimental import pallas as pl


def kernel(x, edge_index, batch, graph_features, W1, b1, W2, b2, Wlin, blin):
    raise NotImplementedError("write your pallas kernel here")


# speedup vs baseline: 9.5359x; 9.5359x over previous
import functools

import jax
import jax.numpy as jnp
from jax import lax
from jax.experimental import pallas as pl
from jax.experimental.pallas import tpu as pltpu
from jax.experimental.pallas import tpu_sc as plsc

NC = 2
NS = 16
NW = NC * NS
CH = 128


def _sc_mesh():
    return plsc.VectorSubcoreMesh(core_axis_name="c", subcore_axis_name="s",
                                  num_cores=NC, num_subcores=NS)


def _make_deg_kernel(E_pad, N_pad):
    n_chunks = E_pad // (NW * CH)
    rows = N_pad // NS

    @functools.partial(
        pl.kernel,
        mesh=_sc_mesh(),
        out_type=jax.ShapeDtypeStruct((NC, N_pad), jnp.float32),
        scratch_types=[
            pltpu.VMEM((CH,), jnp.int32),
            pltpu.VMEM((CH,), jnp.float32),
            pltpu.VMEM((rows,), jnp.float32),
            pltpu.VMEM_SHARED((N_pad,), jnp.float32),
        ],
    )
    def deg_kernel(dst_hbm, out_hbm, idx_v, ones_v, zero_v, shared):
        c = lax.axis_index("c")
        s = lax.axis_index("s")
        wid = c * NS + s
        for j in range(CH // 16):
            ones_v[pl.ds(j * 16, 16)] = jnp.ones((16,), jnp.float32)
        for j in range(rows // 16):
            zero_v[pl.ds(j * 16, 16)] = jnp.zeros((16,), jnp.float32)
        pltpu.sync_copy(zero_v, shared.at[pl.ds(s * rows, rows)])
        plsc.subcore_barrier()

        def body(i, carry):
            base = pl.multiple_of((wid * n_chunks + i) * CH, CH)
            pltpu.sync_copy(dst_hbm.at[pl.ds(base, CH)], idx_v)
            pltpu.sync_copy(ones_v, shared.at[idx_v], add=True)
            return carry

        lax.fori_loop(0, n_chunks, body, 0)
        plsc.subcore_barrier()
        pltpu.sync_copy(shared.at[pl.ds(s * rows, rows)],
                        out_hbm.at[c, pl.ds(s * rows, rows)])

    return deg_kernel


def _make_agg_kernel(E_pad, N_pad, D):
    n_chunks = E_pad // (NW * CH)
    rows = N_pad // NS

    @functools.partial(
        pl.kernel,
        mesh=_sc_mesh(),
        out_type=jax.ShapeDtypeStruct((NC, N_pad, D), jnp.float32),
        scratch_types=[
            pltpu.VMEM((CH,), jnp.int32),
            pltpu.VMEM((CH,), jnp.int32),
            pltpu.VMEM((CH, D), jnp.float32),
            pltpu.SemaphoreType.DMA,
            pltpu.VMEM_SHARED((N_pad, D), jnp.float32),
        ],
    )
    def agg_kernel(val_hbm, src_hbm, dst_hbm, zeros_hbm, out_hbm,
                   sidx_v, didx_v, rows_v, sem, shared):
        c = lax.axis_index("c")
        s = lax.axis_index("s")
        wid = c * NS + s
        pltpu.sync_copy(zeros_hbm.at[pl.ds(s * rows, rows)],
                        shared.at[pl.ds(s * rows, rows)])
        plsc.subcore_barrier()

        def body(i, carry):
            base = pl.multiple_of((wid * n_chunks + i) * CH, CH)
            pltpu.sync_copy(src_hbm.at[pl.ds(base, CH)], sidx_v)
            pltpu.sync_copy(dst_hbm.at[pl.ds(base, CH)], didx_v)
            pltpu.async_copy(val_hbm.at[sidx_v], rows_v, sem).wait()
            pltpu.sync_copy(rows_v, shared.at[didx_v], add=True)
            return carry

        lax.fori_loop(0, n_chunks, body, 0)
        plsc.subcore_barrier()
        pltpu.sync_copy(shared.at[pl.ds(s * rows, rows)],
                        out_hbm.at[c, pl.ds(s * rows, rows)])

    return agg_kernel


def _tc_prep(degt, xp, R):
    N_pad, F = xp.shape
    nb = N_pad // R

    def body(degt_ref, x_ref, dinv_ref, val0_ref):
        deg = degt_ref[:, 0:1] + degt_ref[:, 1:2] + 1.0
        dinv = lax.rsqrt(deg)
        dinv_ref[...] = dinv
        val0_ref[...] = x_ref[...] * dinv

    return pl.pallas_call(
        body,
        grid=(nb,),
        in_specs=[pl.BlockSpec((R, 2), lambda r: (r, 0)),
                  pl.BlockSpec((R, F), lambda r: (r, 0))],
        out_specs=[pl.BlockSpec((R, 1), lambda r: (r, 0)),
                   pl.BlockSpec((R, F), lambda r: (r, 0))],
        out_shape=[jax.ShapeDtypeStruct((N_pad, 1), jnp.float32),
                   jax.ShapeDtypeStruct((N_pad, F), jnp.float32)],
    )(degt, xp)


def _tc_layer1(acc0, val0, dinv, W1, b1, R):
    N_pad, F = val0.shape
    H = W1.shape[1]
    Dh = H // 2
    nb = N_pad // R

    def body(acc_ref, val0_ref, dinv_ref, w_ref, b_ref, outa_ref, outb_ref):
        acc = acc_ref[0] + acc_ref[1]
        dinv = dinv_ref[...]
        u1 = dinv * (acc + val0_ref[...])
        h1 = jnp.dot(u1, w_ref[...], preferred_element_type=jnp.float32)
        h1 = jnp.maximum(h1 + b_ref[...], 0.0)
        val1 = dinv * h1
        outa_ref[...] = val1[:, :Dh]
        outb_ref[...] = val1[:, Dh:]

    return pl.pallas_call(
        body,
        grid=(nb,),
        in_specs=[pl.BlockSpec((2, R, F), lambda r: (0, r, 0)),
                  pl.BlockSpec((R, F), lambda r: (r, 0)),
                  pl.BlockSpec((R, 1), lambda r: (r, 0)),
                  pl.BlockSpec((F, H), lambda r: (0, 0)),
                  pl.BlockSpec((1, H), lambda r: (0, 0))],
        out_specs=[pl.BlockSpec((R, Dh), lambda r: (r, 0)),
                   pl.BlockSpec((R, Dh), lambda r: (r, 0))],
        out_shape=[jax.ShapeDtypeStruct((N_pad, Dh), jnp.float32),
                   jax.ShapeDtypeStruct((N_pad, Dh), jnp.float32)],
    )(acc0, val0, dinv, W1, b1)


def _tc_final(acc1a, acc1b, val1a, val1b, dinv, batch3, gf,
              W2a, W2b, b2, Wl_top, Wl_bot, blin, R, G, C):
    N_pad, Dh = val1a.shape
    H = 2 * Dh
    nb = N_pad // R

    def body(aa_ref, ab_ref, va_ref, vb_ref, dinv_ref, b3_ref, gf_ref,
             w2a_ref, w2b_ref, b2_ref, wlt_ref, wlb_ref, bl_ref,
             out_ref, pooled_scr, cnt_scr):
        r = pl.program_id(0)

        @pl.when(r == 0)
        def _init():
            pooled_scr[...] = jnp.zeros_like(pooled_scr)
            cnt_scr[...] = jnp.zeros_like(cnt_scr)

        dinv = dinv_ref[...]
        u2a = dinv * (aa_ref[0] + aa_ref[1] + va_ref[...])
        u2b = dinv * (ab_ref[0] + ab_ref[1] + vb_ref[...])
        h2 = (jnp.dot(u2a, w2a_ref[...], preferred_element_type=jnp.float32)
              + jnp.dot(u2b, w2b_ref[...], preferred_element_type=jnp.float32))
        h2 = jnp.maximum(h2 + b2_ref[...], 0.0)
        seg = b3_ref[0]
        ids = lax.broadcasted_iota(jnp.int32, (G, R), 0)
        oh = (ids == seg).astype(jnp.float32)
        pooled_scr[...] += jnp.dot(oh, h2, preferred_element_type=jnp.float32)
        cnt_scr[...] += jnp.sum(oh, axis=1, keepdims=True)

        @pl.when(r == nb - 1)
        def _fin():
            pooled = pooled_scr[...] / jnp.maximum(cnt_scr[...], 1.0)
            out_ref[...] = (
                jnp.dot(pooled, wlt_ref[...], preferred_element_type=jnp.float32)
                + jnp.dot(gf_ref[...], wlb_ref[...],
                          preferred_element_type=jnp.float32)
                + bl_ref[...])

    GF = gf.shape[1]
    return pl.pallas_call(
        body,
        grid=(nb,),
        in_specs=[pl.BlockSpec((2, R, Dh), lambda r: (0, r, 0)),
                  pl.BlockSpec((2, R, Dh), lambda r: (0, r, 0)),
                  pl.BlockSpec((R, Dh), lambda r: (r, 0)),
                  pl.BlockSpec((R, Dh), lambda r: (r, 0)),
                  pl.BlockSpec((R, 1), lambda r: (r, 0)),
                  pl.BlockSpec((1, 1, R), lambda r: (r, 0, 0)),
                  pl.BlockSpec((G, GF), lambda r: (0, 0)),
                  pl.BlockSpec((Dh, H), lambda r: (0, 0)),
                  pl.BlockSpec((Dh, H), lambda r: (0, 0)),
                  pl.BlockSpec((1, H), lambda r: (0, 0)),
                  pl.BlockSpec((H, C), lambda r: (0, 0)),
                  pl.BlockSpec((GF, C), lambda r: (0, 0)),
                  pl.BlockSpec((1, C), lambda r: (0, 0))],
        out_specs=pl.BlockSpec((G, C), lambda r: (0, 0)),
        out_shape=jax.ShapeDtypeStruct((G, C), jnp.float32),
        scratch_shapes=[pltpu.VMEM((G, H), jnp.float32),
                        pltpu.VMEM((G, 1), jnp.float32)],
    )(acc1a, acc1b, val1a, val1b, dinv, batch3, gf,
      W2a, W2b, b2, Wl_top, Wl_bot, blin)


def kernel(x, edge_index, batch, graph_features, W1, b1, W2, b2, Wlin, blin):
    N, F = x.shape
    E = edge_index.shape[1]
    H = W1.shape[1]
    G, GF = graph_features.shape
    C = Wlin.shape[1]
    Dh = H // 2

    R = 16 * ((N + 1 + NS * 16 - 1) // (NS * 16))
    N_pad = NS * R
    nb = N_pad // R
    E_pad = ((E + NW * CH - 1) // (NW * CH)) * (NW * CH)

    src = jnp.pad(edge_index[0], (0, E_pad - E), constant_values=N)
    dst = jnp.pad(edge_index[1], (0, E_pad - E), constant_values=N)
    xp = jnp.pad(x, ((0, N_pad - N), (0, 0)))
    batch3 = jnp.pad(batch, (0, N_pad - N),
                     constant_values=G).reshape(nb, 1, R)
    zeros2d = jnp.zeros((N_pad, Dh), jnp.float32)

    degp = _make_deg_kernel(E_pad, N_pad)(dst)
    degt = degp.T
    dinv, val0 = _tc_prep(degt, xp, R)

    agg_f = _make_agg_kernel(E_pad, N_pad, F)
    acc0 = agg_f(val0, src, dst, jnp.zeros((N_pad, F), jnp.float32))

    val1a, val1b = _tc_layer1(acc0, val0, dinv, W1, b1.reshape(1, H), R)

    agg_h = _make_agg_kernel(E_pad, N_pad, Dh)
    acc1a = agg_h(val1a, src, dst, zeros2d)
    acc1b = agg_h(val1b, src, dst, zeros2d)

    out = _tc_final(acc1a, acc1b, val1a, val1b, dinv, batch3,
                    graph_features, W2[:Dh], W2[Dh:], b2.reshape(1, H),
                    Wlin[:H], Wlin[H:], blin.reshape(1, C), R, G, C)
    return out


# speedup vs baseline: 10.7338x; 1.1256x over previous
import functools

import jax
import jax.numpy as jnp
from jax import lax
from jax.experimental import pallas as pl
from jax.experimental.pallas import tpu as pltpu
from jax.experimental.pallas import tpu_sc as plsc

NC = 2
NS = 16
NW = NC * NS
CH = 128


def _sc_mesh():
    return plsc.VectorSubcoreMesh(core_axis_name="c", subcore_axis_name="s",
                                  num_cores=NC, num_subcores=NS)


def _make_deg_kernel(n_chunks, N_pad):
    rows = N_pad // NS

    @functools.partial(
        pl.kernel,
        mesh=_sc_mesh(),
        out_type=jax.ShapeDtypeStruct((NC, N_pad), jnp.float32),
        scratch_types=[
            pltpu.VMEM((n_chunks, CH), jnp.int32),
            pltpu.VMEM((CH,), jnp.float32),
            pltpu.VMEM((rows,), jnp.float32),
            pltpu.SemaphoreType.DMA,
            pltpu.VMEM_SHARED((N_pad,), jnp.float32),
        ],
    )
    def deg_kernel(dst_hbm, out_hbm, idx_v, ones_v, zero_v, sem, shared):
        c = lax.axis_index("c")
        s = lax.axis_index("s")
        wid = c * NS + s
        for j in range(CH // 16):
            ones_v[pl.ds(j * 16, 16)] = jnp.ones((16,), jnp.float32)
        for j in range(rows // 16):
            zero_v[pl.ds(j * 16, 16)] = jnp.zeros((16,), jnp.float32)
        pltpu.sync_copy(dst_hbm.at[wid], idx_v)
        pltpu.sync_copy(zero_v, shared.at[pl.ds(s * rows, rows)])
        plsc.subcore_barrier()

        def fire(i, carry):
            pltpu.async_copy(ones_v, shared.at[idx_v.at[i]], sem, add=True)
            return carry

        lax.fori_loop(0, n_chunks, fire, 0)

        def drain(i, carry):
            pltpu.make_async_copy(ones_v, shared.at[idx_v.at[0]], sem).wait()
            return carry

        lax.fori_loop(0, n_chunks, drain, 0)
        plsc.subcore_barrier()
        pltpu.sync_copy(shared.at[pl.ds(s * rows, rows)],
                        out_hbm.at[c, pl.ds(s * rows, rows)])

    return deg_kernel


SB = 20


def _make_agg_kernel(n_chunks, N_pad, D, idx_per_core):
    rows = N_pad // NS
    nsup = n_chunks // SB

    @functools.partial(
        pl.kernel,
        mesh=_sc_mesh(),
        out_type=jax.ShapeDtypeStruct((NC, N_pad, D), jnp.float32),
        scratch_types=[
            pltpu.VMEM((2, SB, CH), jnp.int32),
            pltpu.VMEM((CH, D), jnp.float32),
            pltpu.VMEM((CH, D), jnp.float32),
            pltpu.SemaphoreType.DMA,
            pltpu.SemaphoreType.DMA,
            pltpu.VMEM_SHARED((N_pad, D), jnp.float32),
        ],
    )
    def agg_kernel(val_hbm, idx_hbm, zeros_hbm, out_hbm,
                   ibuf, bufa, bufb, sema, semb, shared):
        c = lax.axis_index("c")
        s = lax.axis_index("s")
        pltpu.sync_copy(zeros_hbm.at[pl.ds(s * rows, rows)],
                        shared.at[pl.ds(s * rows, rows)])
        plsc.subcore_barrier()

        def wait_gather(buf, sem):
            pltpu.make_async_copy(val_hbm.at[pl.ds(0, CH)], buf, sem).wait()

        def super_body(j, carry):
            if idx_per_core:
                pltpu.sync_copy(idx_hbm.at[c, s, j], ibuf)
            else:
                pltpu.sync_copy(idx_hbm.at[c * NS + s, j], ibuf)
            pltpu.async_copy(val_hbm.at[ibuf.at[0, 0]], bufa, sema)

            def body(k, carry2):
                pltpu.async_copy(val_hbm.at[ibuf.at[0, 2 * k + 1]], bufb, semb)
                wait_gather(bufa, sema)
                pltpu.sync_copy(bufa, shared.at[ibuf.at[1, 2 * k]], add=True)

                @pl.when(k < SB // 2 - 1)
                def _next():
                    pltpu.async_copy(val_hbm.at[ibuf.at[0, 2 * k + 2]],
                                     bufa, sema)

                wait_gather(bufb, semb)
                pltpu.sync_copy(bufb, shared.at[ibuf.at[1, 2 * k + 1]],
                                add=True)
                return carry2

            lax.fori_loop(0, SB // 2, body, 0)
            return carry

        lax.fori_loop(0, nsup, super_body, 0)
        plsc.subcore_barrier()
        pltpu.sync_copy(shared.at[pl.ds(s * rows, rows)],
                        out_hbm.at[c, pl.ds(s * rows, rows)])

    return agg_kernel


def _tc_prep(degt, xp, R):
    N_pad, F = xp.shape
    nb = N_pad // R

    def body(degt_ref, x_ref, dinv_ref, val0_ref):
        deg = degt_ref[:, 0:1] + degt_ref[:, 1:2] + 1.0
        dinv = lax.rsqrt(deg)
        dinv_ref[...] = dinv
        val0_ref[...] = x_ref[...] * dinv

    return pl.pallas_call(
        body,
        grid=(nb,),
        in_specs=[pl.BlockSpec((R, 2), lambda r: (r, 0)),
                  pl.BlockSpec((R, F), lambda r: (r, 0))],
        out_specs=[pl.BlockSpec((R, 1), lambda r: (r, 0)),
                   pl.BlockSpec((R, F), lambda r: (r, 0))],
        out_shape=[jax.ShapeDtypeStruct((N_pad, 1), jnp.float32),
                   jax.ShapeDtypeStruct((N_pad, F), jnp.float32)],
    )(degt, xp)


def _tc_layer1(acc0, val0, dinv, W1, b1, R):
    N_pad, F = val0.shape
    H = W1.shape[1]
    Dh = H // 2
    nb = N_pad // R

    def body(acc_ref, val0_ref, dinv_ref, w_ref, b_ref, out_ref):
        acc = acc_ref[0] + acc_ref[1]
        dinv = dinv_ref[...]
        u1 = dinv * (acc + val0_ref[...])
        h1 = jnp.dot(u1, w_ref[...], preferred_element_type=jnp.float32)
        h1 = jnp.maximum(h1 + b_ref[...], 0.0)
        val1 = dinv * h1
        out_ref[0] = val1[:, :Dh]
        out_ref[1] = val1[:, Dh:]

    return pl.pallas_call(
        body,
        grid=(nb,),
        in_specs=[pl.BlockSpec((2, R, F), lambda r: (0, r, 0)),
                  pl.BlockSpec((R, F), lambda r: (r, 0)),
                  pl.BlockSpec((R, 1), lambda r: (r, 0)),
                  pl.BlockSpec((F, H), lambda r: (0, 0)),
                  pl.BlockSpec((1, H), lambda r: (0, 0))],
        out_specs=pl.BlockSpec((2, R, Dh), lambda r: (0, r, 0)),
        out_shape=jax.ShapeDtypeStruct((2, N_pad, Dh), jnp.float32),
    )(acc0, val0, dinv, W1, b1)


def _tc_final(acc1, val1s, dinv, batch3, gf,
              W2a, W2b, b2, Wl_top, Wl_bot, blin, R, G, C):
    _, N_pad, Dh = val1s.shape
    H = 2 * Dh
    nb = N_pad // R

    def body(acc_ref, val_ref, dinv_ref, b3_ref, gf_ref,
             w2a_ref, w2b_ref, b2_ref, wlt_ref, wlb_ref, bl_ref,
             out_ref, pooled_scr, cnt_scr):
        r = pl.program_id(0)

        @pl.when(r == 0)
        def _init():
            pooled_scr[...] = jnp.zeros_like(pooled_scr)
            cnt_scr[...] = jnp.zeros_like(cnt_scr)

        dinv = dinv_ref[...]
        u2a = dinv * (acc_ref[0] + val_ref[0])
        u2b = dinv * (acc_ref[1] + val_ref[1])
        h2 = (jnp.dot(u2a, w2a_ref[...], preferred_element_type=jnp.float32)
              + jnp.dot(u2b, w2b_ref[...], preferred_element_type=jnp.float32))
        h2 = jnp.maximum(h2 + b2_ref[...], 0.0)
        seg = b3_ref[0]
        ids = lax.broadcasted_iota(jnp.int32, (pooled_scr.shape[0], R), 0)
        oh = (ids == seg).astype(jnp.float32)
        pooled_scr[...] += jnp.dot(oh, h2, preferred_element_type=jnp.float32)
        cnt_scr[...] += jnp.sum(oh, axis=1, keepdims=True)

        @pl.when(r == nb - 1)
        def _fin():
            pooled = pooled_scr[...] / jnp.maximum(cnt_scr[...], 1.0)
            out_ref[...] = (
                jnp.dot(pooled, wlt_ref[...], preferred_element_type=jnp.float32)
                + jnp.dot(gf_ref[...], wlb_ref[...],
                          preferred_element_type=jnp.float32)
                + bl_ref[...])

    GF = gf.shape[1]
    return pl.pallas_call(
        body,
        grid=(nb,),
        in_specs=[pl.BlockSpec((2, R, Dh), lambda r: (0, r, 0)),
                  pl.BlockSpec((2, R, Dh), lambda r: (0, r, 0)),
                  pl.BlockSpec((R, 1), lambda r: (r, 0)),
                  pl.BlockSpec((1, 1, R), lambda r: (r, 0, 0)),
                  pl.BlockSpec((G, GF), lambda r: (0, 0)),
                  pl.BlockSpec((Dh, H), lambda r: (0, 0)),
                  pl.BlockSpec((Dh, H), lambda r: (0, 0)),
                  pl.BlockSpec((1, H), lambda r: (0, 0)),
                  pl.BlockSpec((H, C), lambda r: (0, 0)),
                  pl.BlockSpec((GF, C), lambda r: (0, 0)),
                  pl.BlockSpec((1, C), lambda r: (0, 0))],
        out_specs=pl.BlockSpec((G, C), lambda r: (0, 0)),
        out_shape=jax.ShapeDtypeStruct((G, C), jnp.float32),
        scratch_shapes=[pltpu.VMEM((G, H), jnp.float32),
                        pltpu.VMEM((G, 1), jnp.float32)],
    )(acc1, val1s, dinv, batch3, gf,
      W2a, W2b, b2, Wl_top, Wl_bot, blin)


def kernel(x, edge_index, batch, graph_features, W1, b1, W2, b2, Wlin, blin):
    N, F = x.shape
    E = edge_index.shape[1]
    H = W1.shape[1]
    G, GF = graph_features.shape
    C = Wlin.shape[1]
    Dh = H // 2

    R = 16 * ((N + 1 + NS * 16 - 1) // (NS * 16))
    N_pad = NS * R
    nb = N_pad // R
    Eq = NW * CH * SB
    E_pad = ((E + Eq - 1) // Eq) * Eq
    nch1 = E_pad // (NW * CH)
    nch2 = E_pad // (NS * CH)
    nsup1 = nch1 // SB
    nsup2 = nch2 // SB

    src = jnp.pad(edge_index[0], (0, E_pad - E), constant_values=N)
    dst = jnp.pad(edge_index[1], (0, E_pad - E), constant_values=N)
    dst1 = dst.reshape(NW, nch1, CH)
    idx1 = jnp.stack([src.reshape(NW, nsup1, SB, CH),
                      dst.reshape(NW, nsup1, SB, CH)], axis=2)
    src2 = jnp.stack([src, src + N_pad]).reshape(NC, NS, nsup2, SB, CH)
    dst2 = jnp.broadcast_to(dst.reshape(1, NS, nsup2, SB, CH),
                            (NC, NS, nsup2, SB, CH))
    idx2 = jnp.stack([src2, dst2], axis=3)
    xp = jnp.pad(x, ((0, N_pad - N), (0, 0)))
    batch3 = jnp.pad(batch, (0, N_pad - N),
                     constant_values=G).reshape(nb, 1, R)
    zeros1 = jnp.zeros((N_pad, F), jnp.float32)
    zeros2 = jnp.zeros((N_pad, Dh), jnp.float32)

    degp = _make_deg_kernel(nch1, N_pad)(dst1)
    degt = degp.T
    dinv, val0 = _tc_prep(degt, xp, R)

    acc0 = _make_agg_kernel(nch1, N_pad, F, False)(
        val0, idx1, zeros1)

    val1s = _tc_layer1(acc0, val0, dinv, W1, b1.reshape(1, H), R)
    val1_flat = val1s.reshape(2 * N_pad, Dh)

    acc1 = _make_agg_kernel(nch2, N_pad, Dh, True)(
        val1_flat, idx2, zeros2)

    out = _tc_final(acc1, val1s, dinv, batch3,
                    graph_features, W2[:Dh], W2[Dh:], b2.reshape(1, H),
                    Wlin[:H], Wlin[H:], blin.reshape(1, C), R, G, C)
    return out


# speedup vs baseline: 28.4926x; 2.6545x over previous
import functools

import jax
import jax.numpy as jnp
from jax import lax
from jax.experimental import pallas as pl
from jax.experimental.pallas import tpu as pltpu
from jax.experimental.pallas import tpu_sc as plsc

NC = 2
NS = 16
NW = NC * NS
CH = 128


def _sc_mesh():
    return plsc.VectorSubcoreMesh(core_axis_name="c", subcore_axis_name="s",
                                  num_cores=NC, num_subcores=NS)


def _make_deg_kernel(n_chunks, N_pad):
    rows = N_pad // NS

    @functools.partial(
        pl.kernel,
        mesh=_sc_mesh(),
        out_type=jax.ShapeDtypeStruct((NC, N_pad), jnp.float32),
        scratch_types=[
            pltpu.VMEM((n_chunks, CH), jnp.int32),
            pltpu.VMEM((CH,), jnp.float32),
            pltpu.VMEM((rows,), jnp.float32),
            pltpu.SemaphoreType.DMA,
            pltpu.VMEM_SHARED((N_pad,), jnp.float32),
        ],
    )
    def deg_kernel(dst_hbm, out_hbm, idx_v, ones_v, zero_v, sem, shared):
        c = lax.axis_index("c")
        s = lax.axis_index("s")
        wid = c * NS + s
        for j in range(CH // 16):
            ones_v[pl.ds(j * 16, 16)] = jnp.ones((16,), jnp.float32)
        for j in range(rows // 16):
            zero_v[pl.ds(j * 16, 16)] = jnp.zeros((16,), jnp.float32)
        pltpu.sync_copy(dst_hbm.at[wid], idx_v)
        pltpu.sync_copy(zero_v, shared.at[pl.ds(s * rows, rows)])
        plsc.subcore_barrier()

        def fire(i, carry):
            pltpu.async_copy(ones_v, shared.at[idx_v.at[i]], sem, add=True)
            return carry

        lax.fori_loop(0, n_chunks, fire, 0)

        def drain(i, carry):
            pltpu.make_async_copy(ones_v, shared.at[idx_v.at[0]], sem).wait()
            return carry

        lax.fori_loop(0, n_chunks, drain, 0)
        plsc.subcore_barrier()
        pltpu.sync_copy(shared.at[pl.ds(s * rows, rows)],
                        out_hbm.at[c, pl.ds(s * rows, rows)])

    return deg_kernel


SB = 20


def _make_agg_kernel(n_chunks, N_pad, D, idx_per_core):
    rows = N_pad // NS
    nsup = n_chunks // SB

    @functools.partial(
        pl.kernel,
        mesh=_sc_mesh(),
        out_type=jax.ShapeDtypeStruct((NC, N_pad, D), jnp.float32),
        scratch_types=[
            pltpu.VMEM((2, SB, CH), jnp.int32),
            pltpu.VMEM((CH, D), jnp.float32),
            pltpu.VMEM((CH, D), jnp.float32),
            pltpu.SemaphoreType.DMA,
            pltpu.SemaphoreType.DMA,
            pltpu.VMEM_SHARED((N_pad, D), jnp.float32),
        ],
    )
    def agg_kernel(val_hbm, idx_hbm, zeros_hbm, out_hbm,
                   ibuf, bufa, bufb, sema, semb, shared):
        c = lax.axis_index("c")
        s = lax.axis_index("s")
        pltpu.sync_copy(zeros_hbm.at[pl.ds(s * rows, rows)],
                        shared.at[pl.ds(s * rows, rows)])
        plsc.subcore_barrier()

        def wait_gather(buf, sem):
            pltpu.make_async_copy(val_hbm.at[pl.ds(0, CH)], buf, sem).wait()

        def super_body(j, carry):
            if idx_per_core:
                pltpu.sync_copy(idx_hbm.at[c, s, j], ibuf)
            else:
                pltpu.sync_copy(idx_hbm.at[c * NS + s, j], ibuf)
            pltpu.async_copy(val_hbm.at[ibuf.at[0, 0]], bufa, sema)

            def body(k, carry2):
                pltpu.async_copy(val_hbm.at[ibuf.at[0, 2 * k + 1]], bufb, semb)
                wait_gather(bufa, sema)
                pltpu.sync_copy(bufa, shared.at[ibuf.at[1, 2 * k]], add=True)

                @pl.when(k < SB // 2 - 1)
                def _next():
                    pltpu.async_copy(val_hbm.at[ibuf.at[0, 2 * k + 2]],
                                     bufa, sema)

                wait_gather(bufb, semb)
                pltpu.sync_copy(bufb, shared.at[ibuf.at[1, 2 * k + 1]],
                                add=True)
                return carry2

            lax.fori_loop(0, SB // 2, body, 0)
            return carry

        lax.fori_loop(0, nsup, super_body, 0)
        plsc.subcore_barrier()
        pltpu.sync_copy(shared.at[pl.ds(s * rows, rows)],
                        out_hbm.at[c, pl.ds(s * rows, rows)])

    return agg_kernel


def _tc_prep(degt, xp, R):
    N_pad, F = xp.shape
    nb = N_pad // R

    def body(degt_ref, x_ref, dinv_ref, val0_ref):
        deg = degt_ref[:, 0:1] + degt_ref[:, 1:2] + 1.0
        dinv = lax.rsqrt(deg)
        dinv_ref[...] = dinv
        val0_ref[...] = x_ref[...] * dinv

    return pl.pallas_call(
        body,
        grid=(nb,),
        in_specs=[pl.BlockSpec((R, 2), lambda r: (r, 0)),
                  pl.BlockSpec((R, F), lambda r: (r, 0))],
        out_specs=[pl.BlockSpec((R, 1), lambda r: (r, 0)),
                   pl.BlockSpec((R, F), lambda r: (r, 0))],
        out_shape=[jax.ShapeDtypeStruct((N_pad, 1), jnp.float32),
                   jax.ShapeDtypeStruct((N_pad, F), jnp.float32)],
    )(degt, xp)


def _tc_layer1(acc0, val0, dinv, W1, b1, R):
    N_pad, F = val0.shape
    H = W1.shape[1]
    Dh = H // 2
    nb = N_pad // R

    def body(acc_ref, val0_ref, dinv_ref, w_ref, b_ref, out_ref):
        acc = acc_ref[0] + acc_ref[1]
        dinv = dinv_ref[...]
        u1 = dinv * (acc + val0_ref[...])
        h1 = jnp.dot(u1, w_ref[...], preferred_element_type=jnp.float32)
        h1 = jnp.maximum(h1 + b_ref[...], 0.0)
        val1 = dinv * h1
        out_ref[0] = val1[:, :Dh]
        out_ref[1] = val1[:, Dh:]

    return pl.pallas_call(
        body,
        grid=(nb,),
        in_specs=[pl.BlockSpec((2, R, F), lambda r: (0, r, 0)),
                  pl.BlockSpec((R, F), lambda r: (r, 0)),
                  pl.BlockSpec((R, 1), lambda r: (r, 0)),
                  pl.BlockSpec((F, H), lambda r: (0, 0)),
                  pl.BlockSpec((1, H), lambda r: (0, 0))],
        out_specs=pl.BlockSpec((2, R, Dh), lambda r: (0, r, 0)),
        out_shape=jax.ShapeDtypeStruct((2, N_pad, Dh), jnp.float32),
    )(acc0, val0, dinv, W1, b1)


def _tc_final(acc1, val1s, dinv, batch3, gf,
              W2a, W2b, b2, Wl_top, Wl_bot, blin, R, G, C):
    _, N_pad, Dh = val1s.shape
    H = 2 * Dh
    nb = N_pad // R

    def body(acc_ref, val_ref, dinv_ref, b3_ref, gf_ref,
             w2a_ref, w2b_ref, b2_ref, wlt_ref, wlb_ref, bl_ref,
             out_ref, pooled_scr, cnt_scr):
        r = pl.program_id(0)

        @pl.when(r == 0)
        def _init():
            pooled_scr[...] = jnp.zeros_like(pooled_scr)
            cnt_scr[...] = jnp.zeros_like(cnt_scr)

        dinv = dinv_ref[...]
        u2a = dinv * (acc_ref[0] + val_ref[0])
        u2b = dinv * (acc_ref[1] + val_ref[1])
        h2 = (jnp.dot(u2a, w2a_ref[...], preferred_element_type=jnp.float32)
              + jnp.dot(u2b, w2b_ref[...], preferred_element_type=jnp.float32))
        h2 = jnp.maximum(h2 + b2_ref[...], 0.0)
        seg = b3_ref[0]
        ids = lax.broadcasted_iota(jnp.int32, (pooled_scr.shape[0], R), 0)
        oh = (ids == seg).astype(jnp.float32)
        pooled_scr[...] += jnp.dot(oh, h2, preferred_element_type=jnp.float32)
        cnt_scr[...] += jnp.sum(oh, axis=1, keepdims=True)

        @pl.when(r == nb - 1)
        def _fin():
            pooled = pooled_scr[...] / jnp.maximum(cnt_scr[...], 1.0)
            out_ref[...] = (
                jnp.dot(pooled, wlt_ref[...], preferred_element_type=jnp.float32)
                + jnp.dot(gf_ref[...], wlb_ref[...],
                          preferred_element_type=jnp.float32)
                + bl_ref[...])

    GF = gf.shape[1]
    return pl.pallas_call(
        body,
        grid=(nb,),
        in_specs=[pl.BlockSpec((2, R, Dh), lambda r: (0, r, 0)),
                  pl.BlockSpec((2, R, Dh), lambda r: (0, r, 0)),
                  pl.BlockSpec((R, 1), lambda r: (r, 0)),
                  pl.BlockSpec((1, 1, R), lambda r: (r, 0, 0)),
                  pl.BlockSpec((G, GF), lambda r: (0, 0)),
                  pl.BlockSpec((Dh, H), lambda r: (0, 0)),
                  pl.BlockSpec((Dh, H), lambda r: (0, 0)),
                  pl.BlockSpec((1, H), lambda r: (0, 0)),
                  pl.BlockSpec((H, C), lambda r: (0, 0)),
                  pl.BlockSpec((GF, C), lambda r: (0, 0)),
                  pl.BlockSpec((1, C), lambda r: (0, 0))],
        out_specs=pl.BlockSpec((G, C), lambda r: (0, 0)),
        out_shape=jax.ShapeDtypeStruct((G, C), jnp.float32),
        scratch_shapes=[pltpu.VMEM((G, H), jnp.float32),
                        pltpu.VMEM((G, 1), jnp.float32)],
    )(acc1, val1s, dinv, batch3, gf,
      W2a, W2b, b2, Wl_top, Wl_bot, blin)


def kernel(x, edge_index, batch, graph_features, W1, b1, W2, b2, Wlin, blin):
    N, F = x.shape
    E = edge_index.shape[1]
    H = W1.shape[1]
    G, GF = graph_features.shape
    C = Wlin.shape[1]
    Dh = H // 2

    R = 16 * ((N + 1 + NS * 16 - 1) // (NS * 16))
    N_pad = NS * R
    nb = N_pad // R
    Eq = NW * CH * SB
    E_pad = ((E + Eq - 1) // Eq) * Eq
    nch1 = E_pad // (NW * CH)
    nch2 = E_pad // (NS * CH)
    nsup1 = nch1 // SB
    nsup2 = nch2 // SB

    pad_rows = N + (jnp.arange(E_pad - E, dtype=jnp.int32) % (N_pad - N))
    src = jnp.concatenate([edge_index[0], pad_rows])
    dst = jnp.concatenate([edge_index[1], pad_rows])
    dst1 = dst.reshape(NW, nch1, CH)
    idx1 = jnp.stack([src.reshape(NW, nsup1, SB, CH),
                      dst.reshape(NW, nsup1, SB, CH)], axis=2)
    src2 = jnp.stack([src, src + N_pad]).reshape(NC, NS, nsup2, SB, CH)
    dst2 = jnp.broadcast_to(dst.reshape(1, NS, nsup2, SB, CH),
                            (NC, NS, nsup2, SB, CH))
    idx2 = jnp.stack([src2, dst2], axis=3)
    xp = jnp.pad(x, ((0, N_pad - N), (0, 0)))
    batch3 = jnp.pad(batch, (0, N_pad - N),
                     constant_values=G).reshape(nb, 1, R)
    zeros1 = jnp.zeros((N_pad, F), jnp.float32)
    zeros2 = jnp.zeros((N_pad, Dh), jnp.float32)

    degp = _make_deg_kernel(nch1, N_pad)(dst1)
    degt = degp.T
    dinv, val0 = _tc_prep(degt, xp, R)

    acc0 = _make_agg_kernel(nch1, N_pad, F, False)(
        val0, idx1, zeros1)

    val1s = _tc_layer1(acc0, val0, dinv, W1, b1.reshape(1, H), R)
    val1_flat = val1s.reshape(2 * N_pad, Dh)

    acc1 = _make_agg_kernel(nch2, N_pad, Dh, True)(
        val1_flat, idx2, zeros2)

    out = _tc_final(acc1, val1s, dinv, batch3,
                    graph_features, W2[:Dh], W2[Dh:], b2.reshape(1, H),
                    Wlin[:H], Wlin[H:], blin.reshape(1, C), R, G, C)
    return out
